# Initial kernel scaffold; baseline (speedup 1.0000x reference)
import jax
import jax.numpy as jnp
from jax.experimental import pallas as pl


def kernel(hidden_states, gate_weight, Wg, Wu, Wd, Wsg, Wsu, Wsd):
    raise NotImplementedError("write your pallas kernel here")


# speedup vs baseline: 12.1161x; 12.1161x over previous
import jax
import jax.numpy as jnp
from jax.experimental import pallas as pl

N_EXP = 64
D_MODEL = 8
D_FF = 16
BLK = 2048


def _moe_block(x_ref, gwT_ref, A_ref, B_ref, C_ref, SgT_ref, SuT_ref, SdT_ref,
               o_ref):
    x = x_ref[...]
    logits = jnp.dot(x, gwT_ref[...], preferred_element_type=jnp.float32)
    m = jnp.max(logits, axis=1, keepdims=True)
    exps = jnp.exp(logits - m)
    w = 1.0 / jnp.sum(exps, axis=1, keepdims=True)
    iota = jax.lax.broadcasted_iota(jnp.int32, logits.shape, 1)
    masked = jnp.where(logits == m, iota, N_EXP)
    first = jnp.min(masked, axis=1, keepdims=True)
    ie512 = jax.lax.broadcasted_iota(jnp.int32, (BLK, N_EXP * D_MODEL), 1)
    xt = jnp.concatenate([x] * N_EXP, axis=1)
    zx = jnp.where(ie512 // D_MODEL == first, xt, 0.0)
    g = jnp.dot(zx, A_ref[...], preferred_element_type=jnp.float32)
    u = jnp.dot(zx, B_ref[...], preferred_element_type=jnp.float32)
    h = (g * jax.nn.sigmoid(g)) * u

    ie1024 = jax.lax.broadcasted_iota(jnp.int32, (BLK, N_EXP * D_FF), 1)
    ht = jnp.concatenate([h] * N_EXP, axis=1)
    zh = jnp.where(ie1024 // D_FF == first, ht * w, 0.0)
    routed = jnp.dot(zh, C_ref[...], preferred_element_type=jnp.float32)

    gs = jnp.dot(x, SgT_ref[...], preferred_element_type=jnp.float32)
    us = jnp.dot(x, SuT_ref[...], preferred_element_type=jnp.float32)
    hs = (gs * jax.nn.sigmoid(gs)) * us
    shared = jnp.dot(hs, SdT_ref[...], preferred_element_type=jnp.float32)

    o_ref[...] = routed + shared


def kernel(hidden_states, gate_weight, Wg, Wu, Wd, Wsg, Wsu, Wsd):
    Bsz, S, D = hidden_states.shape
    T = Bsz * S
    x2 = hidden_states.reshape(T, D)
    gwT = gate_weight.T
    A = Wg.transpose(0, 2, 1).reshape(N_EXP * D_MODEL, D_FF)
    B = Wu.transpose(0, 2, 1).reshape(N_EXP * D_MODEL, D_FF)
    C = Wd.transpose(0, 2, 1).reshape(N_EXP * D_FF, D_MODEL)

    full = lambda arr: pl.BlockSpec(arr.shape, lambda i: (0, 0))
    out = pl.pallas_call(
        _moe_block,
        grid=(T // BLK,),
        in_specs=[
            pl.BlockSpec((BLK, D_MODEL), lambda i: (i, 0)),
            full(gwT), full(A), full(B), full(C),
            full(Wsg.T), full(Wsu.T), full(Wsd.T),
        ],
        out_specs=pl.BlockSpec((BLK, D_MODEL), lambda i: (i, 0)),
        out_shape=jax.ShapeDtypeStruct((T, D_MODEL), jnp.float32),
    )(x2, gwT, A, B, C, Wsg.T, Wsu.T, Wsd.T)
    return out.reshape(Bsz, S, D)


# speedup vs baseline: 12.4833x; 1.0303x over previous
import jax
import jax.numpy as jnp
import numpy as np
from jax.experimental import pallas as pl

N_EXP = 64
D_MODEL = 8
D_FF = 16
BLK = 2048


def _moe_block(x_ref, M1_ref, Wall_ref, S2_ref, K_ref, o_ref):
    x = x_ref[...]
    t1 = jnp.dot(x, M1_ref[...], preferred_element_type=jnp.float32)
    logits = t1[:, :N_EXP]
    gs = t1[:, N_EXP:N_EXP + D_FF]
    us = t1[:, N_EXP + D_FF:N_EXP + 2 * D_FF]

    m = jnp.max(logits, axis=1, keepdims=True)
    w = 1.0 / jnp.sum(jnp.exp(logits - m), axis=1, keepdims=True)
    iota = jax.lax.broadcasted_iota(jnp.int32, logits.shape, 1)
    masked = jnp.where(logits == m, iota, N_EXP)
    first = jnp.min(masked, axis=1, keepdims=True)
    oh = (iota == first).astype(jnp.float32)

    Wt = jnp.dot(oh, Wall_ref[...], preferred_element_type=jnp.float32)

    xt = jnp.concatenate([x] * D_FF, axis=1)
    pg = Wt[:, :128] * xt
    pu = Wt[:, 128:256] * xt
    gu = jnp.dot(jnp.concatenate([pg, pu], axis=1), S2_ref[...],
                 preferred_element_type=jnp.float32)
    g = gu[:, :D_FF]
    u = gu[:, D_FF:]
    h = (g * jax.nn.sigmoid(g)) * u * w

    ht = jnp.concatenate([h] * D_MODEL, axis=1)
    pd = Wt[:, 256:384] * ht
    hs = (gs * jax.nn.sigmoid(gs)) * us
    o_ref[...] = jnp.dot(jnp.concatenate([pd, hs], axis=1), K_ref[...],
                         preferred_element_type=jnp.float32)


def kernel(hidden_states, gate_weight, Wg, Wu, Wd, Wsg, Wsu, Wsd):
    Bsz, S, D = hidden_states.shape
    T = Bsz * S
    x2 = hidden_states.reshape(T, D)

    M1 = jnp.concatenate([gate_weight.T, Wsg.T, Wsu.T], axis=1)
    Wall = jnp.concatenate(
        [Wg.reshape(N_EXP, 128), Wu.reshape(N_EXP, 128),
         Wd.reshape(N_EXP, 128)], axis=1)
    S8 = np.kron(np.eye(D_FF, dtype=np.float32), np.ones((D_MODEL, 1), np.float32))
    S16 = np.kron(np.eye(D_MODEL, dtype=np.float32), np.ones((D_FF, 1), np.float32))
    S2 = np.zeros((256, 2 * D_FF), np.float32)
    S2[:128, :D_FF] = S8
    S2[128:, D_FF:] = S8
    S2 = jnp.asarray(S2)
    K = jnp.concatenate([jnp.asarray(S16), Wsd.T], axis=0)

    full = lambda arr: pl.BlockSpec(arr.shape, lambda i: (0, 0))
    out = pl.pallas_call(
        _moe_block,
        grid=(T // BLK,),
        in_specs=[
            pl.BlockSpec((BLK, D_MODEL), lambda i: (i, 0)),
            full(M1), full(Wall), full(S2), full(K),
        ],
        out_specs=pl.BlockSpec((BLK, D_MODEL), lambda i: (i, 0)),
        out_shape=jax.ShapeDtypeStruct((T, D_MODEL), jnp.float32),
    )(x2, M1, Wall, S2, K)
    return out.reshape(Bsz, S, D)


# speedup vs baseline: 28.7057x; 2.2995x over previous
import jax
import jax.numpy as jnp
import numpy as np
from jax.experimental import pallas as pl

N_EXP = 64
D_MODEL = 8
D_FF = 16
BLK = 2048


def _moe_block(x_ref, M1_ref, Wall_ref, S2_ref, K_ref, Rx_ref, Rh_ref, o_ref):
    x = x_ref[...]
    t1 = jnp.dot(x, M1_ref[...], preferred_element_type=jnp.float32)
    logits = t1[:, :N_EXP]
    gs = t1[:, N_EXP:N_EXP + D_FF]
    us = t1[:, N_EXP + D_FF:N_EXP + 2 * D_FF]

    m = jnp.max(logits, axis=1, keepdims=True)
    w = 1.0 / jnp.sum(jnp.exp(logits - m), axis=1, keepdims=True)
    iota = jax.lax.broadcasted_iota(jnp.int32, logits.shape, 1)
    masked = jnp.where(logits == m, iota, N_EXP)
    first = jnp.min(masked, axis=1, keepdims=True)
    oh = (iota == first).astype(jnp.float32)

    Wt = jnp.dot(oh, Wall_ref[...], preferred_element_type=jnp.float32)

    xt = jnp.dot(x, Rx_ref[...], preferred_element_type=jnp.float32)
    pg = Wt[:, :128] * xt
    pu = Wt[:, 128:256] * xt
    gu = jnp.dot(jnp.concatenate([pg, pu], axis=1), S2_ref[...],
                 preferred_element_type=jnp.float32)
    g = gu[:, :D_FF]
    u = gu[:, D_FF:]
    h = (g * jax.nn.sigmoid(g)) * u * w

    ht = jnp.dot(h, Rh_ref[...], preferred_element_type=jnp.float32)
    pd = Wt[:, 256:384] * ht
    hs = (gs * jax.nn.sigmoid(gs)) * us
    o_ref[...] = jnp.dot(jnp.concatenate([pd, hs], axis=1), K_ref[...],
                         preferred_element_type=jnp.float32)


def kernel(hidden_states, gate_weight, Wg, Wu, Wd, Wsg, Wsu, Wsd):
    Bsz, S, D = hidden_states.shape
    T = Bsz * S
    x2 = hidden_states.reshape(T, D)

    M1 = jnp.concatenate([gate_weight.T, Wsg.T, Wsu.T], axis=1)
    Wall = jnp.concatenate(
        [Wg.reshape(N_EXP, 128), Wu.reshape(N_EXP, 128),
         Wd.reshape(N_EXP, 128)], axis=1)
    S8 = np.kron(np.eye(D_FF, dtype=np.float32), np.ones((D_MODEL, 1), np.float32))
    S16 = np.kron(np.eye(D_MODEL, dtype=np.float32), np.ones((D_FF, 1), np.float32))
    S2 = np.zeros((256, 2 * D_FF), np.float32)
    S2[:128, :D_FF] = S8
    S2[128:, D_FF:] = S8
    S2 = jnp.asarray(S2)
    K = jnp.concatenate([jnp.asarray(S16), Wsd.T], axis=0)
    Rx = jnp.asarray(np.kron(np.ones((1, D_FF), np.float32),
                             np.eye(D_MODEL, dtype=np.float32)))
    Rh = jnp.asarray(np.kron(np.ones((1, D_MODEL), np.float32),
                             np.eye(D_FF, dtype=np.float32)))

    full = lambda arr: pl.BlockSpec(arr.shape, lambda i: (0, 0))
    out = pl.pallas_call(
        _moe_block,
        grid=(T // BLK,),
        in_specs=[
            pl.BlockSpec((BLK, D_MODEL), lambda i: (i, 0)),
            full(M1), full(Wall), full(S2), full(K), full(Rx), full(Rh),
        ],
        out_specs=pl.BlockSpec((BLK, D_MODEL), lambda i: (i, 0)),
        out_shape=jax.ShapeDtypeStruct((T, D_MODEL), jnp.float32),
    )(x2, M1, Wall, S2, K, Rx, Rh)
    return out.reshape(Bsz, S, D)
